# baseline (speedup 1.0000x reference)
import functools

import jax
import jax.numpy as jnp
from jax import lax
from jax.experimental import pallas as pl
from jax.experimental.pallas import tpu as pltpu
from jax.experimental.pallas import tpu_sc as plsc

N = 10000
E = 320000
NC = 2
NS = 16
NW = NC * NS
CHUNK = 128
RING = 13
GLA = 7
SLAG = 6
KC = 78
NCHUNK = E // CHUNK
LEFT0 = NW * KC
NLEFT = NCHUNK - LEFT0
RPT = N // NS


def _make_agg(H):
    mesh = plsc.VectorSubcoreMesh(core_axis_name="c", subcore_axis_name="s")

    @functools.partial(
        pl.kernel,
        out_type=jax.ShapeDtypeStruct((NC, N, H), jnp.float32),
        mesh=mesh,
        scratch_types=[
            pltpu.VMEM((KC, CHUNK), jnp.int32),
            pltpu.VMEM((KC, CHUNK), jnp.int32),
            pltpu.VMEM((1, CHUNK), jnp.int32),
            pltpu.VMEM((1, CHUNK), jnp.int32),
            pltpu.VMEM((RING, CHUNK, H), jnp.float32),
            pltpu.VMEM_SHARED((N, H), jnp.float32),
            pltpu.SemaphoreType.DMA,
            pltpu.SemaphoreType.DMA,
        ],
        compiler_params=pltpu.CompilerParams(use_tc_tiling_on_sc=False),
    )
    def agg(hs_hbm, rows_hbm, cols_hbm, zeros_hbm, out_hbm,
            row_v, col_v, lrow_v, lcol_v, msg_v, acc, gsem, ssem):
        c = lax.axis_index("c")
        s = lax.axis_index("s")
        wid = c * NS + s
        pltpu.sync_copy(rows_hbm.at[pl.ds(wid * KC, KC)], row_v)
        pltpu.sync_copy(cols_hbm.at[pl.ds(wid * KC, KC)], col_v)

        @pl.when(wid < NLEFT)
        def _():
            pltpu.sync_copy(rows_hbm.at[pl.ds(LEFT0 + wid, 1)], lrow_v)
            pltpu.sync_copy(cols_hbm.at[pl.ds(LEFT0 + wid, 1)], lcol_v)

        pltpu.sync_copy(zeros_hbm, acc.at[pl.ds(s * RPT, RPT)])
        plsc.subcore_barrier()

        for b in range(GLA):
            pltpu.async_copy(hs_hbm.at[row_v.at[b]], msg_v.at[b], gsem)

        def body(j, carry):
            bj = j % RING
            pltpu.make_async_copy(hs_hbm.at[row_v.at[j]],
                                  msg_v.at[bj], gsem).wait()
            pltpu.async_copy(msg_v.at[bj], acc.at[col_v.at[j]], ssem,
                             add=True)

            @pl.when(j >= SLAG)
            def _():
                jd = j - SLAG
                pltpu.make_async_copy(msg_v.at[jd % RING],
                                      acc.at[col_v.at[jd]], ssem).wait()

            @pl.when(j + GLA < KC)
            def _():
                jg = j + GLA
                pltpu.async_copy(hs_hbm.at[row_v.at[jg]],
                                 msg_v.at[jg % RING], gsem)
            return carry

        lax.fori_loop(0, KC, body, 0)
        for t in range(KC - SLAG, KC):
            pltpu.make_async_copy(msg_v.at[t % RING], acc.at[col_v.at[t]],
                                  ssem).wait()

        @pl.when(wid < NLEFT)
        def _():
            pltpu.async_copy(hs_hbm.at[lrow_v.at[0]], msg_v.at[0], gsem).wait()
            pltpu.async_copy(msg_v.at[0], acc.at[lcol_v.at[0]], ssem,
                             add=True).wait()

        plsc.subcore_barrier()
        pltpu.sync_copy(acc.at[pl.ds(s * RPT, RPT)],
                        out_hbm.at[c, pl.ds(s * RPT, RPT)])

    return agg


def _make_deg():
    mesh = plsc.VectorSubcoreMesh(core_axis_name="c", subcore_axis_name="s")

    @functools.partial(
        pl.kernel,
        out_type=jax.ShapeDtypeStruct((NC, N, 8), jnp.float32),
        mesh=mesh,
        scratch_types=[
            pltpu.VMEM((KC, CHUNK), jnp.int32),
            pltpu.VMEM((1, CHUNK), jnp.int32),
            pltpu.VMEM((CHUNK, 8), jnp.float32),
            pltpu.VMEM_SHARED((N, 8), jnp.float32),
            pltpu.SemaphoreType.DMA,
        ],
        compiler_params=pltpu.CompilerParams(use_tc_tiling_on_sc=False),
    )
    def deg(ones_hbm, cols_hbm, z8_hbm, out_hbm,
            col_v, lcol_v, ones_v, acc, sem):
        c = lax.axis_index("c")
        s = lax.axis_index("s")
        wid = c * NS + s
        pltpu.sync_copy(cols_hbm.at[pl.ds(wid * KC, KC)], col_v)

        @pl.when(wid < NLEFT)
        def _():
            pltpu.sync_copy(cols_hbm.at[pl.ds(LEFT0 + wid, 1)], lcol_v)

        pltpu.sync_copy(ones_hbm, ones_v)
        pltpu.sync_copy(z8_hbm, acc.at[pl.ds(s * RPT, RPT)])
        plsc.subcore_barrier()

        def body(j, carry):
            pltpu.async_copy(ones_v, acc.at[col_v.at[j]], sem, add=True)

            @pl.when(j >= 12)
            def _():
                pltpu.make_async_copy(ones_v, acc.at[col_v.at[j - 12]],
                                      sem).wait()
            return carry

        lax.fori_loop(0, KC, body, 0)
        for t in range(KC - 12, KC):
            pltpu.make_async_copy(ones_v, acc.at[col_v.at[t]], sem).wait()

        @pl.when(wid < NLEFT)
        def _():
            pltpu.async_copy(ones_v, acc.at[lcol_v.at[0]], sem, add=True).wait()

        plsc.subcore_barrier()
        pltpu.sync_copy(acc.at[pl.ds(s * RPT, RPT)],
                        out_hbm.at[c, pl.ds(s * RPT, RPT)])

    return deg


_agg32 = _make_agg(32)
_agg16 = _make_agg(16)
_deg = _make_deg()


def _mm1_body(x_ref, w_ref, h_ref):
    h = jnp.dot(x_ref[0::4, :], w_ref[0], preferred_element_type=jnp.float32)
    for a in range(1, 4):
        h = h + jnp.dot(x_ref[a::4, :], w_ref[a],
                        preferred_element_type=jnp.float32)
    h_ref[...] = h


def _scale_body(h_ref, dp_ref, k16_ref, f8_ref, e4_ref, hs_ref, d32_ref,
                d16_ref):
    dis8 = lax.rsqrt(dp_ref[0] + dp_ref[1] + 1.0)
    v = jnp.dot(dis8, k16_ref[...], preferred_element_type=jnp.float32)
    d16_ref[0::2, :] = jnp.dot(v[:, 0:8], f8_ref[...],
                               preferred_element_type=jnp.float32)
    d16_ref[1::2, :] = jnp.dot(v[:, 8:16], f8_ref[...],
                               preferred_element_type=jnp.float32)
    for i in range(4):
        d32_ref[i::4, :] = jnp.dot(v[:, 4 * i:4 * (i + 1)], e4_ref[...],
                                   preferred_element_type=jnp.float32)
    hs_ref[...] = d32_ref[...] * h_ref[...]


def _mid1_body(p_ref, hs_ref, d32_ref, d16_ref, b_ref, wlo_ref, whi_ref,
               hsn_ref, out_ref):
    out_ref[...] = (d32_ref[...] * (p_ref[0] + p_ref[1] + hs_ref[...])
                    + b_ref[...])
    m = (jnp.dot(out_ref[0::2, :], wlo_ref[...],
                 preferred_element_type=jnp.float32)
         + jnp.dot(out_ref[1::2, :], whi_ref[...],
                   preferred_element_type=jnp.float32))
    hsn_ref[...] = d16_ref[...] * m


def _mid2_body(p_ref, hs_ref, d16_ref, b_ref, wbd_ref, hsn_ref):
    out = d16_ref[...] * (p_ref[0] + p_ref[1] + hs_ref[...]) + b_ref[...]
    m = jnp.dot(out, wbd_ref[...], preferred_element_type=jnp.float32)
    hsn_ref[...] = d16_ref[...] * m


def _fin_body(p_ref, hs_ref, d16_ref, b_ref, g_ref, o_ref):
    z = d16_ref[...] * (p_ref[0] + p_ref[1] + hs_ref[...]) + b_ref[...]
    m = jnp.max(z, axis=1, keepdims=True)
    e = jnp.exp(z - m)
    s = jnp.dot(e, g_ref[...], preferred_element_type=jnp.float32)
    o_ref[...] = (z - m) - jnp.log(s)


def kernel(x, edge_index, W1, b1, W2, b2, W3, b3):
    cols = edge_index[1].reshape(NCHUNK, CHUNK)
    rows = edge_index[0].reshape(NCHUNK, CHUNK)

    ones8 = jnp.ones((CHUNK, 8), jnp.float32)
    z32 = jnp.zeros((RPT, 32), jnp.float32)
    z16 = jnp.zeros((RPT, 16), jnp.float32)
    z8 = jnp.zeros((RPT, 8), jnp.float32)

    b1p = jnp.tile(b1, 4)[None, :]
    b2p = jnp.tile(b2, 8)[None, :]
    b3p = jnp.tile(b3, 8)[None, :]
    w1p = jnp.zeros((4, 128, 128), jnp.float32)
    for a in range(4):
        w1p = w1p.at[a, :, 32 * a:32 * (a + 1)].set(W1)
    w2bd = jax.scipy.linalg.block_diag(W2, W2, W2, W2)
    zpad = jnp.zeros((128, 64), jnp.float32)
    w2lo = jnp.concatenate([w2bd, zpad], axis=1)
    w2hi = jnp.concatenate([zpad, w2bd], axis=1)
    w3bd = jax.scipy.linalg.block_diag(*([W3] * 8))
    g16 = jnp.kron(jnp.eye(8, dtype=jnp.float32),
                   jnp.ones((16, 16), jnp.float32))
    lane = jnp.arange(128)
    k16 = ((lane[:, None] // 8 == jnp.arange(16)[None, :])
           .astype(jnp.float32) / 8.0)
    f8 = (lane[None, :] // 16 == jnp.arange(8)[:, None]).astype(jnp.float32)
    e4 = (lane[None, :] // 32 == jnp.arange(4)[:, None]).astype(jnp.float32)

    dp = _deg(ones8, cols, z8)
    hp = pl.pallas_call(
        _mm1_body,
        out_shape=jax.ShapeDtypeStruct((N // 4, 128), jnp.float32),
    )(x, w1p)
    hs1p, d32, d16 = pl.pallas_call(
        _scale_body,
        out_shape=(jax.ShapeDtypeStruct((N // 4, 128), jnp.float32),
                   jax.ShapeDtypeStruct((N // 4, 128), jnp.float32),
                   jax.ShapeDtypeStruct((N // 8, 128), jnp.float32)),
    )(hp, dp.reshape(NC, N * 8 // 128, 128), k16, f8, e4)

    p1 = _agg32(hs1p.reshape(N, 32), rows, cols, z32)
    hs2p = pl.pallas_call(
        _mid1_body,
        out_shape=jax.ShapeDtypeStruct((N // 8, 128), jnp.float32),
        scratch_shapes=[pltpu.VMEM((N // 4, 128), jnp.float32)],
    )(p1.reshape(NC, N * 32 // 128, 128), hs1p, d32, d16, b1p, w2lo, w2hi)

    p2 = _agg16(hs2p.reshape(N, 16), rows, cols, z16)
    hs3p = pl.pallas_call(
        _mid2_body,
        out_shape=jax.ShapeDtypeStruct((N // 8, 128), jnp.float32),
    )(p2.reshape(NC, N * 16 // 128, 128), hs2p, d16, b2p, w3bd)

    p3 = _agg16(hs3p.reshape(N, 16), rows, cols, z16)
    outp = pl.pallas_call(
        _fin_body,
        out_shape=jax.ShapeDtypeStruct((N // 8, 128), jnp.float32),
    )(p3.reshape(NC, N * 16 // 128, 128), hs3p, d16, b3p, g16)
    return outp.reshape(N, 16)


# speedup vs baseline: 1.0699x; 1.0699x over previous
import functools

import jax
import jax.numpy as jnp
from jax import lax
from jax.experimental import pallas as pl
from jax.experimental.pallas import tpu as pltpu
from jax.experimental.pallas import tpu_sc as plsc

N = 10000
E = 320000
NC = 2
NS = 16
NW = NC * NS
CHUNK = 128
RING = 13
GLA = 7
SLAG = 6
KC = 78
NCHUNK = E // CHUNK
LEFT0 = NW * KC
NLEFT = NCHUNK - LEFT0
RPT = N // NS


def _make_agg(H):
    mesh = plsc.VectorSubcoreMesh(core_axis_name="c", subcore_axis_name="s")

    @functools.partial(
        pl.kernel,
        out_type=jax.ShapeDtypeStruct((NC, N, H), jnp.float32),
        mesh=mesh,
        scratch_types=[
            pltpu.VMEM((KC, CHUNK), jnp.int32),
            pltpu.VMEM((KC, CHUNK), jnp.int32),
            pltpu.VMEM((1, CHUNK), jnp.int32),
            pltpu.VMEM((1, CHUNK), jnp.int32),
            pltpu.VMEM((RING, CHUNK, H), jnp.float32),
            pltpu.VMEM_SHARED((N, H), jnp.float32),
            pltpu.SemaphoreType.DMA,
            pltpu.SemaphoreType.DMA,
        ],
        compiler_params=pltpu.CompilerParams(use_tc_tiling_on_sc=False),
    )
    def agg(hs_hbm, eidx_hbm, zeros_hbm, out_hbm,
            row_v, col_v, lrow_v, lcol_v, msg_v, acc, gsem, ssem):
        c = lax.axis_index("c")
        s = lax.axis_index("s")
        wid = c * NS + s
        pltpu.sync_copy(eidx_hbm.at[0, pl.ds(wid * KC, KC)], row_v)
        pltpu.sync_copy(eidx_hbm.at[1, pl.ds(wid * KC, KC)], col_v)

        @pl.when(wid < NLEFT)
        def _():
            pltpu.sync_copy(eidx_hbm.at[0, pl.ds(LEFT0 + wid, 1)], lrow_v)
            pltpu.sync_copy(eidx_hbm.at[1, pl.ds(LEFT0 + wid, 1)], lcol_v)

        pltpu.sync_copy(zeros_hbm, acc.at[pl.ds(s * RPT, RPT)])
        plsc.subcore_barrier()

        for b in range(GLA):
            pltpu.async_copy(hs_hbm.at[row_v.at[b]], msg_v.at[b], gsem)

        def body(j, carry):
            bj = j % RING
            pltpu.make_async_copy(hs_hbm.at[row_v.at[j]],
                                  msg_v.at[bj], gsem).wait()
            pltpu.async_copy(msg_v.at[bj], acc.at[col_v.at[j]], ssem,
                             add=True)

            @pl.when(j >= SLAG)
            def _():
                jd = j - SLAG
                pltpu.make_async_copy(msg_v.at[jd % RING],
                                      acc.at[col_v.at[jd]], ssem).wait()

            @pl.when(j + GLA < KC)
            def _():
                jg = j + GLA
                pltpu.async_copy(hs_hbm.at[row_v.at[jg]],
                                 msg_v.at[jg % RING], gsem)
            return carry

        lax.fori_loop(0, KC, body, 0)
        for t in range(KC - SLAG, KC):
            pltpu.make_async_copy(msg_v.at[t % RING], acc.at[col_v.at[t]],
                                  ssem).wait()

        @pl.when(wid < NLEFT)
        def _():
            pltpu.async_copy(hs_hbm.at[lrow_v.at[0]], msg_v.at[0], gsem).wait()
            pltpu.async_copy(msg_v.at[0], acc.at[lcol_v.at[0]], ssem,
                             add=True).wait()

        plsc.subcore_barrier()
        pltpu.sync_copy(acc.at[pl.ds(s * RPT, RPT)],
                        out_hbm.at[c, pl.ds(s * RPT, RPT)])

    return agg


def _make_deg():
    mesh = plsc.VectorSubcoreMesh(core_axis_name="c", subcore_axis_name="s")

    @functools.partial(
        pl.kernel,
        out_type=jax.ShapeDtypeStruct((NC, N, 8), jnp.float32),
        mesh=mesh,
        scratch_types=[
            pltpu.VMEM((KC, CHUNK), jnp.int32),
            pltpu.VMEM((1, CHUNK), jnp.int32),
            pltpu.VMEM((CHUNK, 8), jnp.float32),
            pltpu.VMEM_SHARED((N, 8), jnp.float32),
            pltpu.SemaphoreType.DMA,
        ],
        compiler_params=pltpu.CompilerParams(use_tc_tiling_on_sc=False),
    )
    def deg(ones_hbm, eidx_hbm, z8_hbm, out_hbm,
            col_v, lcol_v, ones_v, acc, sem):
        c = lax.axis_index("c")
        s = lax.axis_index("s")
        wid = c * NS + s
        pltpu.sync_copy(eidx_hbm.at[1, pl.ds(wid * KC, KC)], col_v)

        @pl.when(wid < NLEFT)
        def _():
            pltpu.sync_copy(eidx_hbm.at[1, pl.ds(LEFT0 + wid, 1)], lcol_v)

        pltpu.sync_copy(ones_hbm, ones_v)
        pltpu.sync_copy(z8_hbm, acc.at[pl.ds(s * RPT, RPT)])
        plsc.subcore_barrier()

        def body(j, carry):
            pltpu.async_copy(ones_v, acc.at[col_v.at[j]], sem, add=True)

            @pl.when(j >= 12)
            def _():
                pltpu.make_async_copy(ones_v, acc.at[col_v.at[j - 12]],
                                      sem).wait()
            return carry

        lax.fori_loop(0, KC, body, 0)
        for t in range(KC - 12, KC):
            pltpu.make_async_copy(ones_v, acc.at[col_v.at[t]], sem).wait()

        @pl.when(wid < NLEFT)
        def _():
            pltpu.async_copy(ones_v, acc.at[lcol_v.at[0]], sem, add=True).wait()

        plsc.subcore_barrier()
        pltpu.sync_copy(acc.at[pl.ds(s * RPT, RPT)],
                        out_hbm.at[c, pl.ds(s * RPT, RPT)])

    return deg


_agg32 = _make_agg(32)
_agg16 = _make_agg(16)
_deg = _make_deg()


def _mm1_body(x_ref, w_ref, h_ref):
    h = jnp.dot(x_ref[0::4, :], w_ref[0], preferred_element_type=jnp.float32)
    for a in range(1, 4):
        h = h + jnp.dot(x_ref[a::4, :], w_ref[a],
                        preferred_element_type=jnp.float32)
    h_ref[...] = h


def _scale_body(h_ref, dp_ref, k16_ref, f8_ref, e4_ref, hs_ref, d32_ref,
                d16_ref):
    dis8 = lax.rsqrt(dp_ref[0] + dp_ref[1] + 1.0)
    v = jnp.dot(dis8, k16_ref[...], preferred_element_type=jnp.float32)
    d16_ref[0::2, :] = jnp.dot(v[:, 0:8], f8_ref[...],
                               preferred_element_type=jnp.float32)
    d16_ref[1::2, :] = jnp.dot(v[:, 8:16], f8_ref[...],
                               preferred_element_type=jnp.float32)
    for i in range(4):
        d32_ref[i::4, :] = jnp.dot(v[:, 4 * i:4 * (i + 1)], e4_ref[...],
                                   preferred_element_type=jnp.float32)
    hs_ref[...] = d32_ref[...] * h_ref[...]


def _mid1_body(p_ref, hs_ref, d32_ref, d16_ref, b_ref, wlo_ref, whi_ref,
               hsn_ref, out_ref):
    out_ref[...] = (d32_ref[...] * (p_ref[0] + p_ref[1] + hs_ref[...])
                    + b_ref[...])
    m = (jnp.dot(out_ref[0::2, :], wlo_ref[...],
                 preferred_element_type=jnp.float32)
         + jnp.dot(out_ref[1::2, :], whi_ref[...],
                   preferred_element_type=jnp.float32))
    hsn_ref[...] = d16_ref[...] * m


def _mid2_body(p_ref, hs_ref, d16_ref, b_ref, wbd_ref, hsn_ref):
    out = d16_ref[...] * (p_ref[0] + p_ref[1] + hs_ref[...]) + b_ref[...]
    m = jnp.dot(out, wbd_ref[...], preferred_element_type=jnp.float32)
    hsn_ref[...] = d16_ref[...] * m


def _fin_body(p_ref, hs_ref, d16_ref, b_ref, g_ref, o_ref):
    z = d16_ref[...] * (p_ref[0] + p_ref[1] + hs_ref[...]) + b_ref[...]
    m = jnp.max(z, axis=1, keepdims=True)
    e = jnp.exp(z - m)
    s = jnp.dot(e, g_ref[...], preferred_element_type=jnp.float32)
    o_ref[...] = (z - m) - jnp.log(s)


def kernel(x, edge_index, W1, b1, W2, b2, W3, b3):
    eidx = edge_index.reshape(2, NCHUNK, CHUNK)

    ones8 = jnp.ones((CHUNK, 8), jnp.float32)
    z32 = jnp.zeros((RPT, 32), jnp.float32)
    z16 = jnp.zeros((RPT, 16), jnp.float32)
    z8 = jnp.zeros((RPT, 8), jnp.float32)

    b1p = jnp.tile(b1, 4)[None, :]
    b2p = jnp.tile(b2, 8)[None, :]
    b3p = jnp.tile(b3, 8)[None, :]
    w1p = jnp.zeros((4, 128, 128), jnp.float32)
    for a in range(4):
        w1p = w1p.at[a, :, 32 * a:32 * (a + 1)].set(W1)
    w2bd = jax.scipy.linalg.block_diag(W2, W2, W2, W2)
    zpad = jnp.zeros((128, 64), jnp.float32)
    w2lo = jnp.concatenate([w2bd, zpad], axis=1)
    w2hi = jnp.concatenate([zpad, w2bd], axis=1)
    w3bd = jax.scipy.linalg.block_diag(*([W3] * 8))
    g16 = jnp.kron(jnp.eye(8, dtype=jnp.float32),
                   jnp.ones((16, 16), jnp.float32))
    lane = jnp.arange(128)
    k16 = ((lane[:, None] // 8 == jnp.arange(16)[None, :])
           .astype(jnp.float32) / 8.0)
    f8 = (lane[None, :] // 16 == jnp.arange(8)[:, None]).astype(jnp.float32)
    e4 = (lane[None, :] // 32 == jnp.arange(4)[:, None]).astype(jnp.float32)

    dp = _deg(ones8, eidx, z8)
    hp = pl.pallas_call(
        _mm1_body,
        out_shape=jax.ShapeDtypeStruct((N // 4, 128), jnp.float32),
    )(x, w1p)
    hs1p, d32, d16 = pl.pallas_call(
        _scale_body,
        out_shape=(jax.ShapeDtypeStruct((N // 4, 128), jnp.float32),
                   jax.ShapeDtypeStruct((N // 4, 128), jnp.float32),
                   jax.ShapeDtypeStruct((N // 8, 128), jnp.float32)),
    )(hp, dp.reshape(NC, N * 8 // 128, 128), k16, f8, e4)

    p1 = _agg32(hs1p.reshape(N, 32), eidx, z32)
    hs2p = pl.pallas_call(
        _mid1_body,
        out_shape=jax.ShapeDtypeStruct((N // 8, 128), jnp.float32),
        scratch_shapes=[pltpu.VMEM((N // 4, 128), jnp.float32)],
    )(p1.reshape(NC, N * 32 // 128, 128), hs1p, d32, d16, b1p, w2lo, w2hi)

    p2 = _agg16(hs2p.reshape(N, 16), eidx, z16)
    hs3p = pl.pallas_call(
        _mid2_body,
        out_shape=jax.ShapeDtypeStruct((N // 8, 128), jnp.float32),
    )(p2.reshape(NC, N * 16 // 128, 128), hs2p, d16, b2p, w3bd)

    p3 = _agg16(hs3p.reshape(N, 16), eidx, z16)
    outp = pl.pallas_call(
        _fin_body,
        out_shape=jax.ShapeDtypeStruct((N // 8, 128), jnp.float32),
    )(p3.reshape(NC, N * 16 // 128, 128), hs3p, d16, b3p, g16)
    return outp.reshape(N, 16)


# speedup vs baseline: 1.1339x; 1.0598x over previous
import functools

import jax
import jax.numpy as jnp
from jax import lax
from jax.experimental import pallas as pl
from jax.experimental.pallas import tpu as pltpu
from jax.experimental.pallas import tpu_sc as plsc

N = 10000
E = 320000
NC = 2
NS = 16
NW = NC * NS
CHUNK = 128
RING = 22
GLA = 11
SLAG = 11
KC = 78
NCHUNK = E // CHUNK
LEFT0 = NW * KC
NLEFT = NCHUNK - LEFT0
RPT = N // NS


def _make_agg(H):
    mesh = plsc.VectorSubcoreMesh(core_axis_name="c", subcore_axis_name="s")

    @functools.partial(
        pl.kernel,
        out_type=jax.ShapeDtypeStruct((NC, N, H), jnp.float32),
        mesh=mesh,
        scratch_types=[
            pltpu.VMEM((KC, CHUNK), jnp.int32),
            pltpu.VMEM((KC, CHUNK), jnp.int32),
            pltpu.VMEM((1, CHUNK), jnp.int32),
            pltpu.VMEM((1, CHUNK), jnp.int32),
            pltpu.VMEM((RING, CHUNK, H), jnp.float32),
            pltpu.VMEM_SHARED((N, H), jnp.float32),
            pltpu.SemaphoreType.DMA,
            pltpu.SemaphoreType.DMA,
        ],
        compiler_params=pltpu.CompilerParams(use_tc_tiling_on_sc=False),
    )
    def agg(hs_hbm, eidx_hbm, zeros_hbm, out_hbm,
            row_v, col_v, lrow_v, lcol_v, msg_v, acc, gsem, ssem):
        c = lax.axis_index("c")
        s = lax.axis_index("s")
        wid = c * NS + s
        pltpu.sync_copy(eidx_hbm.at[0, pl.ds(wid * KC, KC)], row_v)
        pltpu.sync_copy(eidx_hbm.at[1, pl.ds(wid * KC, KC)], col_v)

        @pl.when(wid < NLEFT)
        def _():
            pltpu.sync_copy(eidx_hbm.at[0, pl.ds(LEFT0 + wid, 1)], lrow_v)
            pltpu.sync_copy(eidx_hbm.at[1, pl.ds(LEFT0 + wid, 1)], lcol_v)

        pltpu.sync_copy(zeros_hbm, acc.at[pl.ds(s * RPT, RPT)])
        plsc.subcore_barrier()

        for b in range(GLA):
            pltpu.async_copy(hs_hbm.at[row_v.at[b]], msg_v.at[b], gsem)

        def body(j, carry):
            bj = j % RING
            pltpu.make_async_copy(hs_hbm.at[row_v.at[j]],
                                  msg_v.at[bj], gsem).wait()
            pltpu.async_copy(msg_v.at[bj], acc.at[col_v.at[j]], ssem,
                             add=True)

            @pl.when(j >= SLAG)
            def _():
                jd = j - SLAG
                pltpu.make_async_copy(msg_v.at[jd % RING],
                                      acc.at[col_v.at[jd]], ssem).wait()

            @pl.when(j + GLA < KC)
            def _():
                jg = j + GLA
                pltpu.async_copy(hs_hbm.at[row_v.at[jg]],
                                 msg_v.at[jg % RING], gsem)
            return carry

        lax.fori_loop(0, KC, body, 0)
        for t in range(KC - SLAG, KC):
            pltpu.make_async_copy(msg_v.at[t % RING], acc.at[col_v.at[t]],
                                  ssem).wait()

        @pl.when(wid < NLEFT)
        def _():
            pltpu.async_copy(hs_hbm.at[lrow_v.at[0]], msg_v.at[0], gsem).wait()
            pltpu.async_copy(msg_v.at[0], acc.at[lcol_v.at[0]], ssem,
                             add=True).wait()

        plsc.subcore_barrier()
        pltpu.sync_copy(acc.at[pl.ds(s * RPT, RPT)],
                        out_hbm.at[c, pl.ds(s * RPT, RPT)])

    return agg


def _make_deg():
    mesh = plsc.VectorSubcoreMesh(core_axis_name="c", subcore_axis_name="s")

    @functools.partial(
        pl.kernel,
        out_type=jax.ShapeDtypeStruct((NC, N, 8), jnp.float32),
        mesh=mesh,
        scratch_types=[
            pltpu.VMEM((KC, CHUNK), jnp.int32),
            pltpu.VMEM((1, CHUNK), jnp.int32),
            pltpu.VMEM((CHUNK, 8), jnp.float32),
            pltpu.VMEM_SHARED((N, 8), jnp.float32),
            pltpu.SemaphoreType.DMA,
        ],
        compiler_params=pltpu.CompilerParams(use_tc_tiling_on_sc=False),
    )
    def deg(ones_hbm, eidx_hbm, z8_hbm, out_hbm,
            col_v, lcol_v, ones_v, acc, sem):
        c = lax.axis_index("c")
        s = lax.axis_index("s")
        wid = c * NS + s
        pltpu.sync_copy(eidx_hbm.at[1, pl.ds(wid * KC, KC)], col_v)

        @pl.when(wid < NLEFT)
        def _():
            pltpu.sync_copy(eidx_hbm.at[1, pl.ds(LEFT0 + wid, 1)], lcol_v)

        pltpu.sync_copy(ones_hbm, ones_v)
        pltpu.sync_copy(z8_hbm, acc.at[pl.ds(s * RPT, RPT)])
        plsc.subcore_barrier()

        def body(j, carry):
            pltpu.async_copy(ones_v, acc.at[col_v.at[j]], sem, add=True)

            @pl.when(j >= 12)
            def _():
                pltpu.make_async_copy(ones_v, acc.at[col_v.at[j - 12]],
                                      sem).wait()
            return carry

        lax.fori_loop(0, KC, body, 0)
        for t in range(KC - 12, KC):
            pltpu.make_async_copy(ones_v, acc.at[col_v.at[t]], sem).wait()

        @pl.when(wid < NLEFT)
        def _():
            pltpu.async_copy(ones_v, acc.at[lcol_v.at[0]], sem, add=True).wait()

        plsc.subcore_barrier()
        pltpu.sync_copy(acc.at[pl.ds(s * RPT, RPT)],
                        out_hbm.at[c, pl.ds(s * RPT, RPT)])

    return deg


_agg32 = _make_agg(32)
_agg16 = _make_agg(16)
_deg = _make_deg()


def _mm1_body(x_ref, w_ref, h_ref):
    h = jnp.dot(x_ref[0::4, :], w_ref[0], preferred_element_type=jnp.float32)
    for a in range(1, 4):
        h = h + jnp.dot(x_ref[a::4, :], w_ref[a],
                        preferred_element_type=jnp.float32)
    h_ref[...] = h


def _scale_body(h_ref, dp_ref, k16_ref, f8_ref, e4_ref, hs_ref, d32_ref,
                d16_ref):
    dis8 = lax.rsqrt(dp_ref[0] + dp_ref[1] + 1.0)
    v = jnp.dot(dis8, k16_ref[...], preferred_element_type=jnp.float32)
    d16_ref[0::2, :] = jnp.dot(v[:, 0:8], f8_ref[...],
                               preferred_element_type=jnp.float32)
    d16_ref[1::2, :] = jnp.dot(v[:, 8:16], f8_ref[...],
                               preferred_element_type=jnp.float32)
    for i in range(4):
        d32_ref[i::4, :] = jnp.dot(v[:, 4 * i:4 * (i + 1)], e4_ref[...],
                                   preferred_element_type=jnp.float32)
    hs_ref[...] = d32_ref[...] * h_ref[...]


def _mid1_body(p_ref, hs_ref, d32_ref, d16_ref, b_ref, wlo_ref, whi_ref,
               hsn_ref, out_ref):
    out_ref[...] = (d32_ref[...] * (p_ref[0] + p_ref[1] + hs_ref[...])
                    + b_ref[...])
    m = (jnp.dot(out_ref[0::2, :], wlo_ref[...],
                 preferred_element_type=jnp.float32)
         + jnp.dot(out_ref[1::2, :], whi_ref[...],
                   preferred_element_type=jnp.float32))
    hsn_ref[...] = d16_ref[...] * m


def _mid2_body(p_ref, hs_ref, d16_ref, b_ref, wbd_ref, hsn_ref):
    out = d16_ref[...] * (p_ref[0] + p_ref[1] + hs_ref[...]) + b_ref[...]
    m = jnp.dot(out, wbd_ref[...], preferred_element_type=jnp.float32)
    hsn_ref[...] = d16_ref[...] * m


def _fin_body(p_ref, hs_ref, d16_ref, b_ref, g_ref, o_ref):
    z = d16_ref[...] * (p_ref[0] + p_ref[1] + hs_ref[...]) + b_ref[...]
    m = jnp.max(z, axis=1, keepdims=True)
    e = jnp.exp(z - m)
    s = jnp.dot(e, g_ref[...], preferred_element_type=jnp.float32)
    o_ref[...] = (z - m) - jnp.log(s)


def kernel(x, edge_index, W1, b1, W2, b2, W3, b3):
    eidx = edge_index.reshape(2, NCHUNK, CHUNK)

    ones8 = jnp.ones((CHUNK, 8), jnp.float32)
    z32 = jnp.zeros((RPT, 32), jnp.float32)
    z16 = jnp.zeros((RPT, 16), jnp.float32)
    z8 = jnp.zeros((RPT, 8), jnp.float32)

    b1p = jnp.tile(b1, 4)[None, :]
    b2p = jnp.tile(b2, 8)[None, :]
    b3p = jnp.tile(b3, 8)[None, :]
    w1p = jnp.zeros((4, 128, 128), jnp.float32)
    for a in range(4):
        w1p = w1p.at[a, :, 32 * a:32 * (a + 1)].set(W1)
    w2bd = jax.scipy.linalg.block_diag(W2, W2, W2, W2)
    zpad = jnp.zeros((128, 64), jnp.float32)
    w2lo = jnp.concatenate([w2bd, zpad], axis=1)
    w2hi = jnp.concatenate([zpad, w2bd], axis=1)
    w3bd = jax.scipy.linalg.block_diag(*([W3] * 8))
    g16 = jnp.kron(jnp.eye(8, dtype=jnp.float32),
                   jnp.ones((16, 16), jnp.float32))
    lane = jnp.arange(128)
    k16 = ((lane[:, None] // 8 == jnp.arange(16)[None, :])
           .astype(jnp.float32) / 8.0)
    f8 = (lane[None, :] // 16 == jnp.arange(8)[:, None]).astype(jnp.float32)
    e4 = (lane[None, :] // 32 == jnp.arange(4)[:, None]).astype(jnp.float32)

    dp = _deg(ones8, eidx, z8)
    hp = pl.pallas_call(
        _mm1_body,
        out_shape=jax.ShapeDtypeStruct((N // 4, 128), jnp.float32),
    )(x, w1p)
    hs1p, d32, d16 = pl.pallas_call(
        _scale_body,
        out_shape=(jax.ShapeDtypeStruct((N // 4, 128), jnp.float32),
                   jax.ShapeDtypeStruct((N // 4, 128), jnp.float32),
                   jax.ShapeDtypeStruct((N // 8, 128), jnp.float32)),
    )(hp, dp.reshape(NC, N * 8 // 128, 128), k16, f8, e4)

    p1 = _agg32(hs1p.reshape(N, 32), eidx, z32)
    hs2p = pl.pallas_call(
        _mid1_body,
        out_shape=jax.ShapeDtypeStruct((N // 8, 128), jnp.float32),
        scratch_shapes=[pltpu.VMEM((N // 4, 128), jnp.float32)],
    )(p1.reshape(NC, N * 32 // 128, 128), hs1p, d32, d16, b1p, w2lo, w2hi)

    p2 = _agg16(hs2p.reshape(N, 16), eidx, z16)
    hs3p = pl.pallas_call(
        _mid2_body,
        out_shape=jax.ShapeDtypeStruct((N // 8, 128), jnp.float32),
    )(p2.reshape(NC, N * 16 // 128, 128), hs2p, d16, b2p, w3bd)

    p3 = _agg16(hs3p.reshape(N, 16), eidx, z16)
    outp = pl.pallas_call(
        _fin_body,
        out_shape=jax.ShapeDtypeStruct((N // 8, 128), jnp.float32),
    )(p3.reshape(NC, N * 16 // 128, 128), hs3p, d16, b3p, g16)
    return outp.reshape(N, 16)


# speedup vs baseline: 1.1384x; 1.0040x over previous
import functools

import jax
import jax.numpy as jnp
from jax import lax
from jax.experimental import pallas as pl
from jax.experimental.pallas import tpu as pltpu
from jax.experimental.pallas import tpu_sc as plsc

N = 10000
E = 320000
NC = 2
NS = 16
NW = NC * NS
CHUNK = 128
RINGS = {32: 22, 16: 40}
KC = 78
NCHUNK = E // CHUNK
LEFT0 = NW * KC
NLEFT = NCHUNK - LEFT0
RPT = N // NS


def _make_agg(H):
    RING = RINGS[H]
    GLA = RING // 2
    SLAG = RING - GLA
    mesh = plsc.VectorSubcoreMesh(core_axis_name="c", subcore_axis_name="s")

    @functools.partial(
        pl.kernel,
        out_type=jax.ShapeDtypeStruct((NC, N, H), jnp.float32),
        mesh=mesh,
        scratch_types=[
            pltpu.VMEM((KC, CHUNK), jnp.int32),
            pltpu.VMEM((KC, CHUNK), jnp.int32),
            pltpu.VMEM((1, CHUNK), jnp.int32),
            pltpu.VMEM((1, CHUNK), jnp.int32),
            pltpu.VMEM((RING, CHUNK, H), jnp.float32),
            pltpu.VMEM_SHARED((N, H), jnp.float32),
            pltpu.SemaphoreType.DMA,
            pltpu.SemaphoreType.DMA,
        ],
        compiler_params=pltpu.CompilerParams(use_tc_tiling_on_sc=False),
    )
    def agg(hs_hbm, eidx_hbm, zeros_hbm, out_hbm,
            row_v, col_v, lrow_v, lcol_v, msg_v, acc, gsem, ssem):
        c = lax.axis_index("c")
        s = lax.axis_index("s")
        wid = c * NS + s
        pltpu.sync_copy(eidx_hbm.at[0, pl.ds(wid * KC, KC)], row_v)
        pltpu.sync_copy(eidx_hbm.at[1, pl.ds(wid * KC, KC)], col_v)

        @pl.when(wid < NLEFT)
        def _():
            pltpu.sync_copy(eidx_hbm.at[0, pl.ds(LEFT0 + wid, 1)], lrow_v)
            pltpu.sync_copy(eidx_hbm.at[1, pl.ds(LEFT0 + wid, 1)], lcol_v)

        pltpu.sync_copy(zeros_hbm, acc.at[pl.ds(s * RPT, RPT)])
        plsc.subcore_barrier()

        for b in range(GLA):
            pltpu.async_copy(hs_hbm.at[row_v.at[b]], msg_v.at[b], gsem)

        def body(j, carry):
            bj = j % RING
            pltpu.make_async_copy(hs_hbm.at[row_v.at[j]],
                                  msg_v.at[bj], gsem).wait()
            pltpu.async_copy(msg_v.at[bj], acc.at[col_v.at[j]], ssem,
                             add=True)

            @pl.when(j >= SLAG)
            def _():
                jd = j - SLAG
                pltpu.make_async_copy(msg_v.at[jd % RING],
                                      acc.at[col_v.at[jd]], ssem).wait()

            @pl.when(j + GLA < KC)
            def _():
                jg = j + GLA
                pltpu.async_copy(hs_hbm.at[row_v.at[jg]],
                                 msg_v.at[jg % RING], gsem)
            return carry

        lax.fori_loop(0, KC, body, 0)
        for t in range(KC - SLAG, KC):
            pltpu.make_async_copy(msg_v.at[t % RING], acc.at[col_v.at[t]],
                                  ssem).wait()

        @pl.when(wid < NLEFT)
        def _():
            pltpu.async_copy(hs_hbm.at[lrow_v.at[0]], msg_v.at[0], gsem).wait()
            pltpu.async_copy(msg_v.at[0], acc.at[lcol_v.at[0]], ssem,
                             add=True).wait()

        plsc.subcore_barrier()
        pltpu.sync_copy(acc.at[pl.ds(s * RPT, RPT)],
                        out_hbm.at[c, pl.ds(s * RPT, RPT)])

    return agg


def _make_deg():
    mesh = plsc.VectorSubcoreMesh(core_axis_name="c", subcore_axis_name="s")

    @functools.partial(
        pl.kernel,
        out_type=jax.ShapeDtypeStruct((NC, N, 8), jnp.float32),
        mesh=mesh,
        scratch_types=[
            pltpu.VMEM((KC, CHUNK), jnp.int32),
            pltpu.VMEM((1, CHUNK), jnp.int32),
            pltpu.VMEM((CHUNK, 8), jnp.float32),
            pltpu.VMEM_SHARED((N, 8), jnp.float32),
            pltpu.SemaphoreType.DMA,
        ],
        compiler_params=pltpu.CompilerParams(use_tc_tiling_on_sc=False),
    )
    def deg(ones_hbm, eidx_hbm, z8_hbm, out_hbm,
            col_v, lcol_v, ones_v, acc, sem):
        c = lax.axis_index("c")
        s = lax.axis_index("s")
        wid = c * NS + s
        pltpu.sync_copy(eidx_hbm.at[1, pl.ds(wid * KC, KC)], col_v)

        @pl.when(wid < NLEFT)
        def _():
            pltpu.sync_copy(eidx_hbm.at[1, pl.ds(LEFT0 + wid, 1)], lcol_v)

        pltpu.sync_copy(ones_hbm, ones_v)
        pltpu.sync_copy(z8_hbm, acc.at[pl.ds(s * RPT, RPT)])
        plsc.subcore_barrier()

        def body(j, carry):
            pltpu.async_copy(ones_v, acc.at[col_v.at[j]], sem, add=True)

            @pl.when(j >= 24)
            def _():
                pltpu.make_async_copy(ones_v, acc.at[col_v.at[j - 24]],
                                      sem).wait()
            return carry

        lax.fori_loop(0, KC, body, 0)
        for t in range(KC - 24, KC):
            pltpu.make_async_copy(ones_v, acc.at[col_v.at[t]], sem).wait()

        @pl.when(wid < NLEFT)
        def _():
            pltpu.async_copy(ones_v, acc.at[lcol_v.at[0]], sem, add=True).wait()

        plsc.subcore_barrier()
        pltpu.sync_copy(acc.at[pl.ds(s * RPT, RPT)],
                        out_hbm.at[c, pl.ds(s * RPT, RPT)])

    return deg


_agg32 = _make_agg(32)
_agg16 = _make_agg(16)
_deg = _make_deg()


def _mm1_body(x_ref, w_ref, h_ref):
    h = jnp.dot(x_ref[0::4, :], w_ref[0], preferred_element_type=jnp.float32)
    for a in range(1, 4):
        h = h + jnp.dot(x_ref[a::4, :], w_ref[a],
                        preferred_element_type=jnp.float32)
    h_ref[...] = h


def _scale_body(h_ref, dp_ref, k16_ref, f8_ref, e4_ref, hs_ref, d32_ref,
                d16_ref):
    dis8 = lax.rsqrt(dp_ref[0] + dp_ref[1] + 1.0)
    v = jnp.dot(dis8, k16_ref[...], preferred_element_type=jnp.float32)
    d16_ref[0::2, :] = jnp.dot(v[:, 0:8], f8_ref[...],
                               preferred_element_type=jnp.float32)
    d16_ref[1::2, :] = jnp.dot(v[:, 8:16], f8_ref[...],
                               preferred_element_type=jnp.float32)
    for i in range(4):
        d32_ref[i::4, :] = jnp.dot(v[:, 4 * i:4 * (i + 1)], e4_ref[...],
                                   preferred_element_type=jnp.float32)
    hs_ref[...] = d32_ref[...] * h_ref[...]


def _mid1_body(p_ref, hs_ref, d32_ref, d16_ref, b_ref, wlo_ref, whi_ref,
               hsn_ref, out_ref):
    out_ref[...] = (d32_ref[...] * (p_ref[0] + p_ref[1] + hs_ref[...])
                    + b_ref[...])
    m = (jnp.dot(out_ref[0::2, :], wlo_ref[...],
                 preferred_element_type=jnp.float32)
         + jnp.dot(out_ref[1::2, :], whi_ref[...],
                   preferred_element_type=jnp.float32))
    hsn_ref[...] = d16_ref[...] * m


def _mid2_body(p_ref, hs_ref, d16_ref, b_ref, wbd_ref, hsn_ref):
    out = d16_ref[...] * (p_ref[0] + p_ref[1] + hs_ref[...]) + b_ref[...]
    m = jnp.dot(out, wbd_ref[...], preferred_element_type=jnp.float32)
    hsn_ref[...] = d16_ref[...] * m


def _fin_body(p_ref, hs_ref, d16_ref, b_ref, g_ref, o_ref):
    z = d16_ref[...] * (p_ref[0] + p_ref[1] + hs_ref[...]) + b_ref[...]
    m = jnp.max(z, axis=1, keepdims=True)
    e = jnp.exp(z - m)
    s = jnp.dot(e, g_ref[...], preferred_element_type=jnp.float32)
    o_ref[...] = (z - m) - jnp.log(s)


def kernel(x, edge_index, W1, b1, W2, b2, W3, b3):
    eidx = edge_index.reshape(2, NCHUNK, CHUNK)

    ones8 = jnp.ones((CHUNK, 8), jnp.float32)
    z32 = jnp.zeros((RPT, 32), jnp.float32)
    z16 = jnp.zeros((RPT, 16), jnp.float32)
    z8 = jnp.zeros((RPT, 8), jnp.float32)

    b1p = jnp.tile(b1, 4)[None, :]
    b2p = jnp.tile(b2, 8)[None, :]
    b3p = jnp.tile(b3, 8)[None, :]
    w1p = jnp.zeros((4, 128, 128), jnp.float32)
    for a in range(4):
        w1p = w1p.at[a, :, 32 * a:32 * (a + 1)].set(W1)
    w2bd = jax.scipy.linalg.block_diag(W2, W2, W2, W2)
    zpad = jnp.zeros((128, 64), jnp.float32)
    w2lo = jnp.concatenate([w2bd, zpad], axis=1)
    w2hi = jnp.concatenate([zpad, w2bd], axis=1)
    w3bd = jax.scipy.linalg.block_diag(*([W3] * 8))
    g16 = jnp.kron(jnp.eye(8, dtype=jnp.float32),
                   jnp.ones((16, 16), jnp.float32))
    lane = jnp.arange(128)
    k16 = ((lane[:, None] // 8 == jnp.arange(16)[None, :])
           .astype(jnp.float32) / 8.0)
    f8 = (lane[None, :] // 16 == jnp.arange(8)[:, None]).astype(jnp.float32)
    e4 = (lane[None, :] // 32 == jnp.arange(4)[:, None]).astype(jnp.float32)

    dp = _deg(ones8, eidx, z8)
    hp = pl.pallas_call(
        _mm1_body,
        out_shape=jax.ShapeDtypeStruct((N // 4, 128), jnp.float32),
    )(x, w1p)
    hs1p, d32, d16 = pl.pallas_call(
        _scale_body,
        out_shape=(jax.ShapeDtypeStruct((N // 4, 128), jnp.float32),
                   jax.ShapeDtypeStruct((N // 4, 128), jnp.float32),
                   jax.ShapeDtypeStruct((N // 8, 128), jnp.float32)),
    )(hp, dp.reshape(NC, N * 8 // 128, 128), k16, f8, e4)

    p1 = _agg32(hs1p.reshape(N, 32), eidx, z32)
    hs2p = pl.pallas_call(
        _mid1_body,
        out_shape=jax.ShapeDtypeStruct((N // 8, 128), jnp.float32),
        scratch_shapes=[pltpu.VMEM((N // 4, 128), jnp.float32)],
    )(p1.reshape(NC, N * 32 // 128, 128), hs1p, d32, d16, b1p, w2lo, w2hi)

    p2 = _agg16(hs2p.reshape(N, 16), eidx, z16)
    hs3p = pl.pallas_call(
        _mid2_body,
        out_shape=jax.ShapeDtypeStruct((N // 8, 128), jnp.float32),
    )(p2.reshape(NC, N * 16 // 128, 128), hs2p, d16, b2p, w3bd)

    p3 = _agg16(hs3p.reshape(N, 16), eidx, z16)
    outp = pl.pallas_call(
        _fin_body,
        out_shape=jax.ShapeDtypeStruct((N // 8, 128), jnp.float32),
    )(p3.reshape(NC, N * 16 // 128, 128), hs3p, d16, b3p, g16)
    return outp.reshape(N, 16)
